# baseline (speedup 1.0000x reference)
import functools

import jax
import jax.numpy as jnp
from jax import lax
from jax.experimental import pallas as pl
from jax.experimental.pallas import tpu as pltpu
from jax.experimental.pallas import tpu_sc as plsc

_LANES = 16
_NSUB = 16
_ROWS = 12500
_CHUNK = 784
_LAST = _ROWS - _CHUNK * (_NSUB - 1)
_ITERS = _CHUNK // 4
_ITERS_LAST = _LAST // 4


def _sum_body(gt_hbm, part_hbm, buf_v, accv_v, sem):
    wid = lax.axis_index("s")
    base = pl.multiple_of(_CHUNK * wid, 8)
    last = _NSUB - 1

    @pl.when(wid < last)
    def _():
        pltpu.async_copy(gt_hbm.at[pl.ds(base, _CHUNK)],
                         buf_v.at[pl.ds(0, _CHUNK)], sem)

    @pl.when(wid == last)
    def _():
        pltpu.async_copy(gt_hbm.at[pl.ds(base, _LAST)],
                         buf_v.at[pl.ds(0, _LAST)], sem)

    @pl.when(wid < last)
    def _():
        pltpu.make_async_copy(gt_hbm.at[pl.ds(base, _CHUNK)],
                              buf_v.at[pl.ds(0, _CHUNK)], sem).wait()

    @pl.when(wid == last)
    def _():
        pltpu.make_async_copy(gt_hbm.at[pl.ds(base, _LAST)],
                              buf_v.at[pl.ds(0, _LAST)], sem).wait()

    zero = jnp.zeros((_LANES,), jnp.int32)
    n_iters = jnp.where(wid == last, _ITERS_LAST, _ITERS)

    def body(i, accs):
        a0, a1, a2, a3 = accs
        r = i * 4
        return (a0 + buf_v[r], a1 + buf_v[r + 1],
                a2 + buf_v[r + 2], a3 + buf_v[r + 3])

    a0, a1, a2, a3 = lax.fori_loop(0, n_iters, body,
                                   (zero, zero, zero, zero))

    accv_v[...] = (a0 + a1) + (a2 + a3)
    pltpu.sync_copy(accv_v, part_hbm.at[wid])


_sum_kernel = functools.partial(
    pl.kernel,
    out_type=jax.ShapeDtypeStruct((_NSUB, _LANES), jnp.int32),
    mesh=plsc.VectorSubcoreMesh(
        core_axis_name="c", subcore_axis_name="s", num_cores=1
    ),
    scratch_types=[
        pltpu.VMEM((_CHUNK, _LANES), jnp.int32),
        pltpu.VMEM((_LANES,), jnp.int32),
        pltpu.SemaphoreType.DMA,
    ],
    compiler_params=pltpu.CompilerParams(use_tc_tiling_on_sc=False),
)(_sum_body)


def _combine_body(part_ref, out_ref):
    s = jnp.sum(part_ref[...])
    loss = jnp.where(s == 0, jnp.float32(0.0), s.astype(jnp.float32))
    out_ref[...] = jnp.full((1, 1), loss, jnp.float32)


_combine_kernel = pl.pallas_call(
    _combine_body,
    out_shape=jax.ShapeDtypeStruct((1, 1), jnp.float32),
)


def kernel(new_t1_feats_list, new_t2_feats_list, gt_labels, edge_idxs,
           mask_trk_gt, edge_batch_idx_offsets):
    del new_t1_feats_list, new_t2_feats_list, edge_idxs
    del mask_trk_gt, edge_batch_idx_offsets
    gt_rows = gt_labels.reshape(_ROWS, _LANES)
    parts = _sum_kernel(gt_rows)
    return _combine_kernel(parts)[0, 0]
